# baseline (speedup 1.0000x reference)
import functools

import jax
import jax.numpy as jnp
from jax import lax
from jax.experimental import pallas as pl
from jax.experimental.pallas import tpu as pltpu
from jax.experimental.pallas import tpu_sc as plsc

N = 10000
E = 320000
D = 128
NC = 8

DA = D + 16
C = 40
NTILES = 32
EPT = E // NTILES
NCH = EPT // C
NPAD = 10240
RPT = NPAD // 16


def _pre_body(x_ref, we1, be1, we2, be2, w1a, w1b, b1i,
              h_ref, hs_ref, hd_ref):
    h = jnp.maximum(x_ref[...] @ we1[...] + be1[...], 0.0) @ we2[...] + be2[...]
    h_ref[...] = h
    hs_ref[...] = h @ w1a[...]
    hd_ref[...] = h @ w1b[...] + b1i[...]


def _tc_pre(x, we1, be1, we2, be2, w1a, w1b, b1i):
    R = 1000
    grid = (N // R,)
    row = pl.BlockSpec((R, D), lambda i: (i, 0))
    full = pl.BlockSpec((D, D), lambda i: (0, 0))
    vec = pl.BlockSpec((1, D), lambda i: (0, 0))
    return pl.pallas_call(
        _pre_body,
        grid=grid,
        in_specs=[row, full, vec, full, vec, full, full, vec],
        out_specs=[row, row, row],
        out_shape=[jax.ShapeDtypeStruct((N, D), jnp.float32)] * 3,
    )(x, we1, be1, we2, be2, w1a, w1b, b1i)


def _edge_body(hs_hbm, hd_hbm, idx_hbm, zero_hbm, out_hbm,
               ibuf, abuf, bbuf, tbuf, aggh,
               gsem0, gsem1, *isems):
    c = lax.axis_index("c")
    s = lax.axis_index("s")
    tid = c * 16 + s
    gsems = (gsem0, gsem1)

    pltpu.sync_copy(zero_hbm, aggh.at[pl.ds(s * RPT, RPT)])

    tail = jnp.where(lax.iota(jnp.int32, 16) == 0, 1.0, 0.0)

    def init_tail(r, carry):
        tbuf[0, r, pl.ds(D, 16)] = tail
        tbuf[1, r, pl.ds(D, 16)] = tail
        return carry

    lax.fori_loop(0, C, init_tail, 0)

    plsc.subcore_barrier()

    pltpu.sync_copy(idx_hbm.at[tid, 0], ibuf.at[0])
    pltpu.sync_copy(idx_hbm.at[tid, 1], ibuf.at[1])
    for jj in range(2, 6):
        pltpu.async_copy(idx_hbm.at[tid, jj], ibuf.at[jj], isems[jj % 8])
    for jj in range(2):
        pltpu.async_copy(hs_hbm.at[ibuf.at[jj, 0]], abuf.at[jj], gsems[jj])
        pltpu.async_copy(hd_hbm.at[ibuf.at[jj, 1]], bbuf.at[jj], gsems[jj])

    def step(j, u, in_loop):
        slot = u % 2
        gsem = gsems[slot]

        pltpu.make_async_copy(hs_hbm.at[ibuf.at[u, 0]], abuf.at[slot],
                              gsem).wait()
        pltpu.make_async_copy(hd_hbm.at[ibuf.at[u, 1]], bbuf.at[slot],
                              gsem).wait()

        @plsc.parallel_loop(0, C, step=1, unroll=4)
        def _compute(r):
            for k in range(D // 16):
                va = abuf[slot, r, pl.ds(k * 16, 16)]
                vb = bbuf[slot, r, pl.ds(k * 16, 16)]
                tbuf[slot, r, pl.ds(k * 16, 16)] = jnp.maximum(va + vb, 0.0)

        pltpu.sync_copy(tbuf.at[slot], aggh.at[ibuf.at[u, 1]], add=True)

        if in_loop:
            nu = (u + 2) % 8
            pltpu.make_async_copy(idx_hbm.at[tid, 0], ibuf.at[nu],
                                  isems[nu]).wait()
            pltpu.async_copy(hs_hbm.at[ibuf.at[nu, 0]], abuf.at[slot], gsem)
            pltpu.async_copy(hd_hbm.at[ibuf.at[nu, 1]], bbuf.at[slot], gsem)

            fu = (u + 6) % 8

            @pl.when(j + 6 < NCH)
            def _():
                pltpu.async_copy(idx_hbm.at[tid, j + 6], ibuf.at[fu],
                                 isems[fu])

    def block8(jb, carry):
        for u in range(8):
            step(jb * 8 + u, u, True)
        return carry

    lax.fori_loop(0, (NCH - 2) // 8, block8, 0)

    step(NCH - 2, (NCH - 2) % 8, False)
    step(NCH - 1, (NCH - 1) % 8, False)

    plsc.subcore_barrier()

    pltpu.sync_copy(aggh.at[pl.ds(s * RPT, RPT)],
                    out_hbm.at[c, pl.ds(s * RPT, RPT)])


@functools.partial(
    pl.kernel,
    out_type=jax.ShapeDtypeStruct((2, NPAD, DA), jnp.float32),
    mesh=plsc.VectorSubcoreMesh(core_axis_name="c", subcore_axis_name="s"),
    compiler_params=pltpu.CompilerParams(use_tc_tiling_on_sc=False),
    scratch_types=[
        pltpu.VMEM((8, 2, C), jnp.int32),
        pltpu.VMEM((2, C, D), jnp.float32),
        pltpu.VMEM((2, C, D), jnp.float32),
        pltpu.VMEM((2, C, DA), jnp.float32),
        pltpu.VMEM_SHARED((NPAD, DA), jnp.float32),
    ] + [pltpu.SemaphoreType.DMA] * 10,
)
def _edge_kernel(hs, hd, idx, zrows, out, *scratch):
    _edge_body(hs, hd, idx, zrows, out, *scratch)


def _post_body(h_ref, part_ref, w2i, b2i, wn1a, wn1b, bn1, wn2, bn2,
               wo1, bo1, wo2, bo2, out_ref):
    p0 = part_ref[0]
    p1 = part_ref[1]
    aggh = p0[:, :D] + p1[:, :D]
    deg = p0[:, D:D + 1] + p1[:, D:D + 1]
    agg = aggh @ w2i[...] + deg * b2i[...]
    h = h_ref[...]
    hn = jnp.maximum(h @ wn1a[...] + agg @ wn1b[...] + bn1[...], 0.0)
    hn = hn @ wn2[...] + bn2[...]
    out_ref[...] = jnp.maximum(hn @ wo1[...] + bo1[...], 0.0) @ wo2[...] + bo2[...]


def _tc_post(h, part, w2i, b2i, wn1a, wn1b, bn1, wn2, bn2, wo1, bo1, wo2, bo2):
    R = 1000
    grid = (N // R,)
    row = pl.BlockSpec((R, D), lambda i: (i, 0))
    prt = pl.BlockSpec((2, R, DA), lambda i: (0, i, 0))
    full = pl.BlockSpec((D, D), lambda i: (0, 0))
    vec = pl.BlockSpec((1, D), lambda i: (0, 0))
    ospec = pl.BlockSpec((R, NC), lambda i: (i, 0))
    ovec = pl.BlockSpec((1, NC), lambda i: (0, 0))
    wout = pl.BlockSpec((D, NC), lambda i: (0, 0))
    return pl.pallas_call(
        _post_body,
        grid=grid,
        in_specs=[row, prt, full, vec, full, full, vec, full, vec,
                  full, vec, wout, ovec],
        out_specs=ospec,
        out_shape=jax.ShapeDtypeStruct((N, NC), jnp.float32),
    )(h, part, w2i, b2i, wn1a, wn1b, bn1, wn2, bn2, wo1, bo1, wo2, bo2)


def kernel(x, edge_index, enc, inter, nodem, outp):
    (we1, be1), (we2, be2) = enc
    (w1i, b1i), (w2i, b2i) = inter
    (wn1, bn1), (wn2, bn2) = nodem
    (wo1, bo1), (wo2, bo2) = outp

    h, hs, hd = _tc_pre(
        x, we1, be1.reshape(1, D), we2, be2.reshape(1, D),
        w1i[:D], w1i[D:], b1i.reshape(1, D))

    ei = edge_index.astype(jnp.int32)
    idx = jnp.stack(
        [ei[0].reshape(NTILES, NCH, C), ei[1].reshape(NTILES, NCH, C)],
        axis=2)
    zrows = jnp.zeros((RPT, DA), jnp.float32)

    part = _edge_kernel(hs, hd, idx, zrows)

    return _tc_post(
        h, part, w2i, b2i.reshape(1, D), wn1[:D], wn1[D:], bn1.reshape(1, D),
        wn2, bn2.reshape(1, D), wo1, bo1.reshape(1, D), wo2, bo2.reshape(1, NC))


# speedup vs baseline: 1.1757x; 1.1757x over previous
import functools

import jax
import jax.numpy as jnp
from jax import lax
from jax.experimental import pallas as pl
from jax.experimental.pallas import tpu as pltpu
from jax.experimental.pallas import tpu_sc as plsc

N = 10000
E = 320000
D = 128
NC = 8

DA = D + 16
C = 40
NTILES = 32
EPT = E // NTILES
NCH = EPT // C
G = 10
NG = NCH // G
NPAD = 10240
RPT = NPAD // 16


def _pre_body(x_ref, we1, be1, we2, be2, w1a, w1b, b1i, h_ref, hsd_ref):
    h = jnp.maximum(x_ref[...] @ we1[...] + be1[...], 0.0) @ we2[...] + be2[...]
    h_ref[...] = h
    hsd_ref[0] = h @ w1b[...] + b1i[...]
    hsd_ref[1] = h @ w1a[...]


def _tc_pre(x, we1, be1, we2, be2, w1a, w1b, b1i):
    R = 1000
    grid = (N // R,)
    row = pl.BlockSpec((R, D), lambda i: (i, 0))
    full = pl.BlockSpec((D, D), lambda i: (0, 0))
    vec = pl.BlockSpec((1, D), lambda i: (0, 0))
    stk = pl.BlockSpec((2, R, D), lambda i: (0, i, 0))
    return pl.pallas_call(
        _pre_body,
        grid=grid,
        in_specs=[row, full, vec, full, vec, full, full, vec],
        out_specs=[row, stk],
        out_shape=[jax.ShapeDtypeStruct((N, D), jnp.float32),
                   jax.ShapeDtypeStruct((2, N, D), jnp.float32)],
    )(x, we1, be1, we2, be2, w1a, w1b, b1i)


def _edge_body(hsd_hbm, idx_hbm, zero_hbm, out_hbm,
               ibufa, ibufb, abuf, tbuf, aggh,
               gsem0, gsem1, isema, isemb):
    c = lax.axis_index("c")
    s = lax.axis_index("s")
    tid = c * 16 + s
    gsems = (gsem0, gsem1)

    pltpu.sync_copy(zero_hbm, aggh.at[pl.ds(s * RPT, RPT)])

    tail = jnp.where(lax.iota(jnp.int32, 16) == 0, 1.0, 0.0)

    def init_tail(r, carry):
        tbuf[r, pl.ds(D, 16)] = tail
        return carry

    lax.fori_loop(0, C, init_tail, 0)

    plsc.subcore_barrier()

    pltpu.sync_copy(idx_hbm.at[tid, pl.ds(0, G)], ibufa)
    for j in range(2):
        pltpu.async_copy(hsd_hbm.at[ibufa.at[j]], abuf.at[j], gsems[j])

    def chunk(g, u, cur, nxt, isem_nxt, issue, wait_idx):
        slot = u % 2
        gsem = gsems[slot]

        pltpu.make_async_copy(hsd_hbm.at[cur.at[u]], abuf.at[slot],
                              gsem).wait()

        @plsc.parallel_loop(0, C, step=1, unroll=4)
        def _compute(r):
            for k in range(D // 16):
                va = abuf[slot, r, pl.ds(k * 16, 16)]
                vb = abuf[slot, C + r, pl.ds(k * 16, 16)]
                tbuf[r, pl.ds(k * 16, 16)] = jnp.maximum(va + vb, 0.0)

        if wait_idx:
            pltpu.make_async_copy(idx_hbm.at[tid, pl.ds(0, G)], nxt,
                                  isem_nxt).wait()

        if issue:
            if u + 2 < G:
                ilist = cur.at[u + 2]
            else:
                ilist = nxt.at[u + 2 - G]
            pltpu.async_copy(hsd_hbm.at[ilist], abuf.at[slot], gsem)

        pltpu.sync_copy(tbuf, aggh.at[cur.at[u, pl.ds(C, C)]], add=True)

    def group(g, cur, nxt, isem_nxt, prefetch, last):
        if prefetch:
            pltpu.async_copy(idx_hbm.at[tid, pl.ds((g + 1) * G, G)], nxt,
                             isem_nxt)
        for u in range(G):
            chunk(g, u, cur, nxt, isem_nxt,
                  issue=not (last and u >= G - 2),
                  wait_idx=(not last) and u == G - 2)

    def pair(t, carry):
        g = t * 2
        group(g, ibufa, ibufb, isemb, True, False)
        group(g + 1, ibufb, ibufa, isema, True, False)
        return carry

    lax.fori_loop(0, (NG - 1) // 2, pair, 0)
    group(NG - 1, ibufa, ibufb, isemb, False, True)

    plsc.subcore_barrier()

    pltpu.sync_copy(aggh.at[pl.ds(s * RPT, RPT)],
                    out_hbm.at[c, pl.ds(s * RPT, RPT)])


@functools.partial(
    pl.kernel,
    out_type=jax.ShapeDtypeStruct((2, NPAD, DA), jnp.float32),
    mesh=plsc.VectorSubcoreMesh(core_axis_name="c", subcore_axis_name="s"),
    compiler_params=pltpu.CompilerParams(use_tc_tiling_on_sc=False),
    scratch_types=[
        pltpu.VMEM((G, 2 * C), jnp.int32),
        pltpu.VMEM((G, 2 * C), jnp.int32),
        pltpu.VMEM((2, 2 * C, D), jnp.float32),
        pltpu.VMEM((C, DA), jnp.float32),
        pltpu.VMEM_SHARED((NPAD, DA), jnp.float32),
    ] + [pltpu.SemaphoreType.DMA] * 4,
)
def _edge_kernel(hsd, idx, zrows, out, *scratch):
    _edge_body(hsd, idx, zrows, out, *scratch)


def _post_body(h_ref, part_ref, w2i, b2i, wn1a, wn1b, bn1, wn2, bn2,
               wo1, bo1, wo2, bo2, out_ref):
    p0 = part_ref[0]
    p1 = part_ref[1]
    aggh = p0[:, :D] + p1[:, :D]
    deg = p0[:, D:D + 1] + p1[:, D:D + 1]
    agg = aggh @ w2i[...] + deg * b2i[...]
    h = h_ref[...]
    hn = jnp.maximum(h @ wn1a[...] + agg @ wn1b[...] + bn1[...], 0.0)
    hn = hn @ wn2[...] + bn2[...]
    out_ref[...] = jnp.maximum(hn @ wo1[...] + bo1[...], 0.0) @ wo2[...] + bo2[...]


def _tc_post(h, part, w2i, b2i, wn1a, wn1b, bn1, wn2, bn2, wo1, bo1, wo2, bo2):
    R = 1000
    grid = (N // R,)
    row = pl.BlockSpec((R, D), lambda i: (i, 0))
    prt = pl.BlockSpec((2, R, DA), lambda i: (0, i, 0))
    full = pl.BlockSpec((D, D), lambda i: (0, 0))
    vec = pl.BlockSpec((1, D), lambda i: (0, 0))
    ospec = pl.BlockSpec((R, NC), lambda i: (i, 0))
    ovec = pl.BlockSpec((1, NC), lambda i: (0, 0))
    wout = pl.BlockSpec((D, NC), lambda i: (0, 0))
    return pl.pallas_call(
        _post_body,
        grid=grid,
        in_specs=[row, prt, full, vec, full, full, vec, full, vec,
                  full, vec, wout, ovec],
        out_specs=ospec,
        out_shape=jax.ShapeDtypeStruct((N, NC), jnp.float32),
    )(h, part, w2i, b2i, wn1a, wn1b, bn1, wn2, bn2, wo1, bo1, wo2, bo2)


def kernel(x, edge_index, enc, inter, nodem, outp):
    (we1, be1), (we2, be2) = enc
    (w1i, b1i), (w2i, b2i) = inter
    (wn1, bn1), (wn2, bn2) = nodem
    (wo1, bo1), (wo2, bo2) = outp

    h, hsd = _tc_pre(
        x, we1, be1.reshape(1, D), we2, be2.reshape(1, D),
        w1i[:D], w1i[D:], b1i.reshape(1, D))

    ei = edge_index.astype(jnp.int32)
    srcp = ei[0].reshape(NTILES, NCH, C) + N
    dstp = ei[1].reshape(NTILES, NCH, C)
    idx = jnp.concatenate([srcp, dstp], axis=2)
    zrows = jnp.zeros((RPT, DA), jnp.float32)

    part = _edge_kernel(hsd.reshape(2 * N, D), idx, zrows)

    return _tc_post(
        h, part, w2i, b2i.reshape(1, D), wn1[:D], wn1[D:], bn1.reshape(1, D),
        wn2, bn2.reshape(1, D), wo1, bo1.reshape(1, D), wo2, bo2.reshape(1, NC))


# speedup vs baseline: 1.1987x; 1.0195x over previous
import functools

import jax
import jax.numpy as jnp
from jax import lax
from jax.experimental import pallas as pl
from jax.experimental.pallas import tpu as pltpu
from jax.experimental.pallas import tpu_sc as plsc

N = 10000
E = 320000
D = 128
NC = 8

DA = D + 16
C = 40
NTILES = 32
EPT = E // NTILES
NCH = EPT // C
G = 10
NG = NCH // G
NPAD = 10240
RPT = NPAD // 16


def _pre_body(x_ref, we1, be1, we2, be2, w1a, w1b, b1i, h_ref, hsd_ref):
    h = jnp.maximum(x_ref[...] @ we1[...] + be1[...], 0.0) @ we2[...] + be2[...]
    h_ref[...] = h
    hsd_ref[0] = (h @ w1b[...] + b1i[...]).astype(jnp.bfloat16)
    hsd_ref[1] = (h @ w1a[...]).astype(jnp.bfloat16)


def _tc_pre(x, we1, be1, we2, be2, w1a, w1b, b1i):
    R = 2000
    grid = (N // R,)
    row = pl.BlockSpec((R, D), lambda i: (i, 0))
    full = pl.BlockSpec((D, D), lambda i: (0, 0))
    vec = pl.BlockSpec((1, D), lambda i: (0, 0))
    stk = pl.BlockSpec((2, R, D), lambda i: (0, i, 0))
    return pl.pallas_call(
        _pre_body,
        grid=grid,
        in_specs=[row, full, vec, full, vec, full, full, vec],
        out_specs=[row, stk],
        out_shape=[jax.ShapeDtypeStruct((N, D), jnp.float32),
                   jax.ShapeDtypeStruct((2, N, D), jnp.bfloat16)],
    )(x, we1, be1, we2, be2, w1a, w1b, b1i)


def _edge_body(hsd_hbm, idx_hbm, zero_hbm, out_hbm,
               ibufa, ibufb, abuf, tbuf, aggh,
               gsem0, gsem1, isema, isemb):
    c = lax.axis_index("c")
    s = lax.axis_index("s")
    tid = c * 16 + s
    gsems = (gsem0, gsem1)

    pltpu.sync_copy(zero_hbm, aggh.at[pl.ds(s * RPT, RPT)])

    tail = jnp.where(lax.iota(jnp.int32, 16) == 0, 1.0, 0.0)

    def init_tail(r, carry):
        tbuf[r, pl.ds(D, 16)] = tail
        return carry

    lax.fori_loop(0, C, init_tail, 0)

    plsc.subcore_barrier()

    pltpu.sync_copy(idx_hbm.at[tid, pl.ds(0, G)], ibufa)
    for j in range(2):
        pltpu.async_copy(hsd_hbm.at[ibufa.at[j]], abuf.at[j], gsems[j])

    def chunk(g, u, cur, nxt, isem_nxt, issue, wait_idx):
        slot = u % 2
        gsem = gsems[slot]

        pltpu.make_async_copy(hsd_hbm.at[cur.at[u]], abuf.at[slot],
                              gsem).wait()

        @plsc.parallel_loop(0, C, step=1, unroll=4)
        def _compute(r):
            for k in range(D // 16):
                va = abuf[slot, r, pl.ds(k * 16, 16)].astype(jnp.float32)
                vb = abuf[slot, C + r, pl.ds(k * 16, 16)].astype(jnp.float32)
                tbuf[r, pl.ds(k * 16, 16)] = jnp.maximum(va + vb, 0.0)

        if wait_idx:
            pltpu.make_async_copy(idx_hbm.at[tid, pl.ds(0, G)], nxt,
                                  isem_nxt).wait()

        if issue:
            if u + 2 < G:
                ilist = cur.at[u + 2]
            else:
                ilist = nxt.at[u + 2 - G]
            pltpu.async_copy(hsd_hbm.at[ilist], abuf.at[slot], gsem)

        pltpu.sync_copy(tbuf, aggh.at[cur.at[u, pl.ds(C, C)]], add=True)

    def group(g, cur, nxt, isem_nxt, prefetch, last):
        if prefetch:
            pltpu.async_copy(idx_hbm.at[tid, pl.ds((g + 1) * G, G)], nxt,
                             isem_nxt)
        for u in range(G):
            chunk(g, u, cur, nxt, isem_nxt,
                  issue=not (last and u >= G - 2),
                  wait_idx=(not last) and u == G - 2)

    def pair(t, carry):
        g = t * 2
        group(g, ibufa, ibufb, isemb, True, False)
        group(g + 1, ibufb, ibufa, isema, True, False)
        return carry

    lax.fori_loop(0, (NG - 1) // 2, pair, 0)
    group(NG - 1, ibufa, ibufb, isemb, False, True)

    plsc.subcore_barrier()

    pltpu.sync_copy(aggh.at[pl.ds(s * RPT, RPT)],
                    out_hbm.at[c, pl.ds(s * RPT, RPT)])


@functools.partial(
    pl.kernel,
    out_type=jax.ShapeDtypeStruct((2, NPAD, DA), jnp.float32),
    mesh=plsc.VectorSubcoreMesh(core_axis_name="c", subcore_axis_name="s"),
    compiler_params=pltpu.CompilerParams(use_tc_tiling_on_sc=False),
    scratch_types=[
        pltpu.VMEM((G, 2 * C), jnp.int32),
        pltpu.VMEM((G, 2 * C), jnp.int32),
        pltpu.VMEM((2, 2 * C, D), jnp.bfloat16),
        pltpu.VMEM((C, DA), jnp.float32),
        pltpu.VMEM_SHARED((NPAD, DA), jnp.float32),
    ] + [pltpu.SemaphoreType.DMA] * 4,
)
def _edge_kernel(hsd, idx, zrows, out, *scratch):
    _edge_body(hsd, idx, zrows, out, *scratch)


def _post_body(h_ref, part_ref, w2i, b2i, wn1a, wn1b, bn1, wn2, bn2,
               wo1, bo1, wo2, bo2, out_ref):
    p0 = part_ref[0]
    p1 = part_ref[1]
    aggh = p0[:, :D] + p1[:, :D]
    deg = p0[:, D:D + 1] + p1[:, D:D + 1]
    agg = aggh @ w2i[...] + deg * b2i[...]
    h = h_ref[...]
    hn = jnp.maximum(h @ wn1a[...] + agg @ wn1b[...] + bn1[...], 0.0)
    hn = hn @ wn2[...] + bn2[...]
    out_ref[...] = jnp.maximum(hn @ wo1[...] + bo1[...], 0.0) @ wo2[...] + bo2[...]


def _tc_post(h, part, w2i, b2i, wn1a, wn1b, bn1, wn2, bn2, wo1, bo1, wo2, bo2):
    R = 1000
    grid = (N // R,)
    row = pl.BlockSpec((R, D), lambda i: (i, 0))
    prt = pl.BlockSpec((2, R, DA), lambda i: (0, i, 0))
    full = pl.BlockSpec((D, D), lambda i: (0, 0))
    vec = pl.BlockSpec((1, D), lambda i: (0, 0))
    ospec = pl.BlockSpec((R, NC), lambda i: (i, 0))
    ovec = pl.BlockSpec((1, NC), lambda i: (0, 0))
    wout = pl.BlockSpec((D, NC), lambda i: (0, 0))
    return pl.pallas_call(
        _post_body,
        grid=grid,
        in_specs=[row, prt, full, vec, full, full, vec, full, vec,
                  full, vec, wout, ovec],
        out_specs=ospec,
        out_shape=jax.ShapeDtypeStruct((N, NC), jnp.float32),
    )(h, part, w2i, b2i, wn1a, wn1b, bn1, wn2, bn2, wo1, bo1, wo2, bo2)


def kernel(x, edge_index, enc, inter, nodem, outp):
    (we1, be1), (we2, be2) = enc
    (w1i, b1i), (w2i, b2i) = inter
    (wn1, bn1), (wn2, bn2) = nodem
    (wo1, bo1), (wo2, bo2) = outp

    h, hsd = _tc_pre(
        x, we1, be1.reshape(1, D), we2, be2.reshape(1, D),
        w1i[:D], w1i[D:], b1i.reshape(1, D))

    ei = edge_index.astype(jnp.int32)
    srcp = ei[0].reshape(NTILES, NCH, C) + N
    dstp = ei[1].reshape(NTILES, NCH, C)
    idx = jnp.concatenate([srcp, dstp], axis=2)
    zrows = jnp.zeros((RPT, DA), jnp.float32)

    part = _edge_kernel(hsd.reshape(2 * N, D), idx, zrows)

    return _tc_post(
        h, part, w2i, b2i.reshape(1, D), wn1[:D], wn1[D:], bn1.reshape(1, D),
        wn2, bn2.reshape(1, D), wo1, bo1.reshape(1, D), wo2, bo2.reshape(1, NC))


# speedup vs baseline: 1.3615x; 1.1358x over previous
import functools

import jax
import jax.numpy as jnp
from jax import lax
from jax.experimental import pallas as pl
from jax.experimental.pallas import tpu as pltpu
from jax.experimental.pallas import tpu_sc as plsc

N = 10000
E = 320000
D = 128
NC = 8

DA = D + 32
C = 40
NTILES = 32
EPT = E // NTILES
NCH = EPT // C
G = 10
NG = NCH // G
NPAD = 10240
RPT = NPAD // 16


def _pre_body(x_ref, we1, be1, we2, be2, w1a, w1b, b1i, h_ref, hsd_ref):
    h = jnp.maximum(x_ref[...] @ we1[...] + be1[...], 0.0) @ we2[...] + be2[...]
    h_ref[...] = h
    hsd_ref[0] = (h @ w1b[...] + b1i[...]).astype(jnp.bfloat16)
    hsd_ref[1] = (h @ w1a[...]).astype(jnp.bfloat16)


def _tc_pre(x, we1, be1, we2, be2, w1a, w1b, b1i):
    R = 2000
    grid = (N // R,)
    row = pl.BlockSpec((R, D), lambda i: (i, 0))
    full = pl.BlockSpec((D, D), lambda i: (0, 0))
    vec = pl.BlockSpec((1, D), lambda i: (0, 0))
    stk = pl.BlockSpec((2, R, D), lambda i: (0, i, 0))
    return pl.pallas_call(
        _pre_body,
        grid=grid,
        in_specs=[row, full, vec, full, vec, full, full, vec],
        out_specs=[row, stk],
        out_shape=[jax.ShapeDtypeStruct((N, D), jnp.float32),
                   jax.ShapeDtypeStruct((2, N, D), jnp.bfloat16)],
    )(x, we1, be1, we2, be2, w1a, w1b, b1i)


def _edge_body(hsd_hbm, idx_hbm, zero_hbm, tinit_hbm, out_hbm,
               ibufa, ibufb, abuf, tbuf, aggh,
               gsem0, gsem1, isema, isemb):
    c = lax.axis_index("c")
    s = lax.axis_index("s")
    tid = c * 16 + s
    gsems = (gsem0, gsem1)

    pltpu.sync_copy(zero_hbm, aggh.at[pl.ds(s * RPT, RPT)])

    pltpu.sync_copy(tinit_hbm, tbuf)

    plsc.subcore_barrier()

    pltpu.sync_copy(idx_hbm.at[tid, pl.ds(0, G)], ibufa)
    for j in range(2):
        pltpu.async_copy(hsd_hbm.at[ibufa.at[j]], abuf.at[j], gsems[j])

    def chunk(g, u, cur, nxt, isem_nxt, issue, wait_idx):
        slot = u % 2
        gsem = gsems[slot]

        pltpu.make_async_copy(hsd_hbm.at[cur.at[u]], abuf.at[slot],
                              gsem).wait()

        @plsc.parallel_loop(0, C, step=1, unroll=4)
        def _compute(r):
            for k in range(D // 32):
                va = abuf[slot, r, pl.ds(k * 32, 32)]
                vb = abuf[slot, C + r, pl.ds(k * 32, 32)]
                tbuf[r, pl.ds(k * 32, 32)] = jnp.maximum(va + vb,
                                                         jnp.bfloat16(0.0))

        if wait_idx:
            pltpu.make_async_copy(idx_hbm.at[tid, pl.ds(0, G)], nxt,
                                  isem_nxt).wait()

        if issue:
            if u + 2 < G:
                ilist = cur.at[u + 2]
            else:
                ilist = nxt.at[u + 2 - G]
            pltpu.async_copy(hsd_hbm.at[ilist], abuf.at[slot], gsem)

        pltpu.sync_copy(tbuf, aggh.at[cur.at[u, pl.ds(C, C)]], add=True)

    def group(g, cur, nxt, isem_nxt, prefetch, last):
        if prefetch:
            pltpu.async_copy(idx_hbm.at[tid, pl.ds((g + 1) * G, G)], nxt,
                             isem_nxt)
        for u in range(G):
            chunk(g, u, cur, nxt, isem_nxt,
                  issue=not (last and u >= G - 2),
                  wait_idx=(not last) and u == G - 2)

    def pair(t, carry):
        g = t * 2
        group(g, ibufa, ibufb, isemb, True, False)
        group(g + 1, ibufb, ibufa, isema, True, False)
        return carry

    lax.fori_loop(0, (NG - 1) // 2, pair, 0)
    group(NG - 1, ibufa, ibufb, isemb, False, True)

    plsc.subcore_barrier()

    pltpu.sync_copy(aggh.at[pl.ds(s * RPT, RPT)],
                    out_hbm.at[c, pl.ds(s * RPT, RPT)])


@functools.partial(
    pl.kernel,
    out_type=jax.ShapeDtypeStruct((2, NPAD, DA), jnp.bfloat16),
    mesh=plsc.VectorSubcoreMesh(core_axis_name="c", subcore_axis_name="s"),
    compiler_params=pltpu.CompilerParams(use_tc_tiling_on_sc=False),
    scratch_types=[
        pltpu.VMEM((G, 2 * C), jnp.int32),
        pltpu.VMEM((G, 2 * C), jnp.int32),
        pltpu.VMEM((2, 2 * C, D), jnp.bfloat16),
        pltpu.VMEM((C, DA), jnp.bfloat16),
        pltpu.VMEM_SHARED((NPAD, DA), jnp.bfloat16),
    ] + [pltpu.SemaphoreType.DMA] * 4,
)
def _edge_kernel(hsd, idx, zrows, tinit, out, *scratch):
    _edge_body(hsd, idx, zrows, tinit, out, *scratch)


def _post_body(h_ref, part_ref, w2i, b2i, wn1a, wn1b, bn1, wn2, bn2,
               wo1, bo1, wo2, bo2, out_ref):
    p0 = part_ref[0].astype(jnp.float32)
    p1 = part_ref[1].astype(jnp.float32)
    aggh = p0[:, :D] + p1[:, :D]
    deg = p0[:, D:D + 1] + p1[:, D:D + 1]
    agg = aggh @ w2i[...] + deg * b2i[...]
    h = h_ref[...]
    hn = jnp.maximum(h @ wn1a[...] + agg @ wn1b[...] + bn1[...], 0.0)
    hn = hn @ wn2[...] + bn2[...]
    out_ref[...] = jnp.maximum(hn @ wo1[...] + bo1[...], 0.0) @ wo2[...] + bo2[...]


def _tc_post(h, part, w2i, b2i, wn1a, wn1b, bn1, wn2, bn2, wo1, bo1, wo2, bo2):
    R = 2000
    grid = (N // R,)
    row = pl.BlockSpec((R, D), lambda i: (i, 0))
    prt = pl.BlockSpec((2, R, DA), lambda i: (0, i, 0))
    full = pl.BlockSpec((D, D), lambda i: (0, 0))
    vec = pl.BlockSpec((1, D), lambda i: (0, 0))
    ospec = pl.BlockSpec((R, NC), lambda i: (i, 0))
    ovec = pl.BlockSpec((1, NC), lambda i: (0, 0))
    wout = pl.BlockSpec((D, NC), lambda i: (0, 0))
    return pl.pallas_call(
        _post_body,
        grid=grid,
        in_specs=[row, prt, full, vec, full, full, vec, full, vec,
                  full, vec, wout, ovec],
        out_specs=ospec,
        out_shape=jax.ShapeDtypeStruct((N, NC), jnp.float32),
    )(h, part, w2i, b2i, wn1a, wn1b, bn1, wn2, bn2, wo1, bo1, wo2, bo2)


def kernel(x, edge_index, enc, inter, nodem, outp):
    (we1, be1), (we2, be2) = enc
    (w1i, b1i), (w2i, b2i) = inter
    (wn1, bn1), (wn2, bn2) = nodem
    (wo1, bo1), (wo2, bo2) = outp

    h, hsd = _tc_pre(
        x, we1, be1.reshape(1, D), we2, be2.reshape(1, D),
        w1i[:D], w1i[D:], b1i.reshape(1, D))

    ei = edge_index.astype(jnp.int32)
    srcp = ei[0].reshape(NTILES, NCH, C) + N
    dstp = ei[1].reshape(NTILES, NCH, C)
    idx = jnp.concatenate([srcp, dstp], axis=2)
    zrows = jnp.zeros((RPT, DA), jnp.bfloat16)
    tinit = jnp.zeros((C, DA), jnp.bfloat16).at[:, D].set(jnp.bfloat16(1.0))

    part = _edge_kernel(hsd.reshape(2 * N, D), idx, zrows, tinit)

    return _tc_post(
        h, part, w2i, b2i.reshape(1, D), wn1[:D], wn1[D:], bn1.reshape(1, D),
        wn2, bn2.reshape(1, D), wo1, bo1.reshape(1, D), wo2, bo2.reshape(1, NC))


# speedup vs baseline: 1.6775x; 1.2321x over previous
import functools

import jax
import jax.numpy as jnp
from jax import lax
from jax.experimental import pallas as pl
from jax.experimental.pallas import tpu as pltpu
from jax.experimental.pallas import tpu_sc as plsc

N = 10000
E = 320000
D = 128
NC = 8

DA = D + 32
C = 200
NTILES = 32
EPT = E // NTILES
NCH = EPT // C
G = 2
NG = NCH // G
NPAD = 10240
RPT = NPAD // 16


def _pre_body(x_ref, we1, be1, we2, be2, w1a, w1b, b1i, h_ref, hsd_ref):
    h = jnp.maximum(x_ref[...] @ we1[...] + be1[...], 0.0) @ we2[...] + be2[...]
    h_ref[...] = h
    hsd_ref[0] = (h @ w1b[...] + b1i[...]).astype(jnp.bfloat16)
    hsd_ref[1] = (h @ w1a[...]).astype(jnp.bfloat16)


def _tc_pre(x, we1, be1, we2, be2, w1a, w1b, b1i):
    R = 2000
    grid = (N // R,)
    row = pl.BlockSpec((R, D), lambda i: (i, 0))
    full = pl.BlockSpec((D, D), lambda i: (0, 0))
    vec = pl.BlockSpec((1, D), lambda i: (0, 0))
    stk = pl.BlockSpec((2, R, D), lambda i: (0, i, 0))
    return pl.pallas_call(
        _pre_body,
        grid=grid,
        in_specs=[row, full, vec, full, vec, full, full, vec],
        out_specs=[row, stk],
        out_shape=[jax.ShapeDtypeStruct((N, D), jnp.float32),
                   jax.ShapeDtypeStruct((2, N, D), jnp.bfloat16)],
    )(x, we1, be1, we2, be2, w1a, w1b, b1i)


def _edge_body(hsd_hbm, idx_hbm, zero_hbm, tinit_hbm, out_hbm,
               ibufa, ibufb, abuf, tbuf, aggh,
               gsem0, gsem1, isema, isemb):
    c = lax.axis_index("c")
    s = lax.axis_index("s")
    tid = c * 16 + s
    gsems = (gsem0, gsem1)

    pltpu.sync_copy(zero_hbm, aggh.at[pl.ds(s * RPT, RPT)])

    pltpu.sync_copy(tinit_hbm, tbuf)

    plsc.subcore_barrier()

    pltpu.sync_copy(idx_hbm.at[tid, pl.ds(0, G)], ibufa)
    for j in range(2):
        pltpu.async_copy(hsd_hbm.at[ibufa.at[j]], abuf.at[j], gsems[j])

    def chunk(g, u, cur, nxt, isem_nxt, issue, wait_idx):
        slot = u % 2
        gsem = gsems[slot]

        pltpu.make_async_copy(hsd_hbm.at[cur.at[u]], abuf.at[slot],
                              gsem).wait()

        @plsc.parallel_loop(0, C, step=1, unroll=4)
        def _compute(r):
            for k in range(D // 32):
                va = abuf[slot, r, pl.ds(k * 32, 32)]
                vb = abuf[slot, C + r, pl.ds(k * 32, 32)]
                tbuf[r, pl.ds(k * 32, 32)] = jnp.maximum(va + vb,
                                                         jnp.bfloat16(0.0))

        if wait_idx:
            pltpu.make_async_copy(idx_hbm.at[tid, pl.ds(0, G)], nxt,
                                  isem_nxt).wait()

        if issue:
            if u + 2 < G:
                ilist = cur.at[u + 2]
            else:
                ilist = nxt.at[u + 2 - G]
            pltpu.async_copy(hsd_hbm.at[ilist], abuf.at[slot], gsem)

        pltpu.sync_copy(tbuf, aggh.at[cur.at[u, pl.ds(C, C)]], add=True)

    def group(g, cur, nxt, isem_nxt, prefetch, last):
        if prefetch:
            pltpu.async_copy(idx_hbm.at[tid, pl.ds((g + 1) * G, G)], nxt,
                             isem_nxt)
        for u in range(G):
            chunk(g, u, cur, nxt, isem_nxt,
                  issue=not (last and u >= G - 2),
                  wait_idx=(not last) and u == G - 2)

    def pair(t, carry):
        g = t * 2
        group(g, ibufa, ibufb, isemb, True, False)
        group(g + 1, ibufb, ibufa, isema, True, False)
        return carry

    lax.fori_loop(0, (NG - 1) // 2, pair, 0)
    group(NG - 1, ibufa, ibufb, isemb, False, True)

    plsc.subcore_barrier()

    pltpu.sync_copy(aggh.at[pl.ds(s * RPT, RPT)],
                    out_hbm.at[c, pl.ds(s * RPT, RPT)])


@functools.partial(
    pl.kernel,
    out_type=jax.ShapeDtypeStruct((2, NPAD, DA), jnp.bfloat16),
    mesh=plsc.VectorSubcoreMesh(core_axis_name="c", subcore_axis_name="s"),
    compiler_params=pltpu.CompilerParams(use_tc_tiling_on_sc=False),
    scratch_types=[
        pltpu.VMEM((G, 2 * C), jnp.int32),
        pltpu.VMEM((G, 2 * C), jnp.int32),
        pltpu.VMEM((2, 2 * C, D), jnp.bfloat16),
        pltpu.VMEM((C, DA), jnp.bfloat16),
        pltpu.VMEM_SHARED((NPAD, DA), jnp.bfloat16),
    ] + [pltpu.SemaphoreType.DMA] * 4,
)
def _edge_kernel(hsd, idx, zrows, tinit, out, *scratch):
    _edge_body(hsd, idx, zrows, tinit, out, *scratch)


def _post_body(h_ref, part_ref, w2i, b2i, wn1a, wn1b, bn1, wn2, bn2,
               wo1, bo1, wo2, bo2, out_ref):
    p0 = part_ref[0].astype(jnp.float32)
    p1 = part_ref[1].astype(jnp.float32)
    aggh = p0[:, :D] + p1[:, :D]
    deg = p0[:, D:D + 1] + p1[:, D:D + 1]
    agg = aggh @ w2i[...] + deg * b2i[...]
    h = h_ref[...]
    hn = jnp.maximum(h @ wn1a[...] + agg @ wn1b[...] + bn1[...], 0.0)
    hn = hn @ wn2[...] + bn2[...]
    out_ref[...] = jnp.maximum(hn @ wo1[...] + bo1[...], 0.0) @ wo2[...] + bo2[...]


def _tc_post(h, part, w2i, b2i, wn1a, wn1b, bn1, wn2, bn2, wo1, bo1, wo2, bo2):
    R = 2000
    grid = (N // R,)
    row = pl.BlockSpec((R, D), lambda i: (i, 0))
    prt = pl.BlockSpec((2, R, DA), lambda i: (0, i, 0))
    full = pl.BlockSpec((D, D), lambda i: (0, 0))
    vec = pl.BlockSpec((1, D), lambda i: (0, 0))
    ospec = pl.BlockSpec((R, NC), lambda i: (i, 0))
    ovec = pl.BlockSpec((1, NC), lambda i: (0, 0))
    wout = pl.BlockSpec((D, NC), lambda i: (0, 0))
    return pl.pallas_call(
        _post_body,
        grid=grid,
        in_specs=[row, prt, full, vec, full, full, vec, full, vec,
                  full, vec, wout, ovec],
        out_specs=ospec,
        out_shape=jax.ShapeDtypeStruct((N, NC), jnp.float32),
    )(h, part, w2i, b2i, wn1a, wn1b, bn1, wn2, bn2, wo1, bo1, wo2, bo2)


def kernel(x, edge_index, enc, inter, nodem, outp):
    (we1, be1), (we2, be2) = enc
    (w1i, b1i), (w2i, b2i) = inter
    (wn1, bn1), (wn2, bn2) = nodem
    (wo1, bo1), (wo2, bo2) = outp

    h, hsd = _tc_pre(
        x, we1, be1.reshape(1, D), we2, be2.reshape(1, D),
        w1i[:D], w1i[D:], b1i.reshape(1, D))

    ei = edge_index.astype(jnp.int32)
    srcp = ei[0].reshape(NTILES, NCH, C) + N
    dstp = ei[1].reshape(NTILES, NCH, C)
    idx = jnp.concatenate([srcp, dstp], axis=2)
    zrows = jnp.zeros((RPT, DA), jnp.bfloat16)
    tinit = jnp.zeros((C, DA), jnp.bfloat16).at[:, D].set(jnp.bfloat16(1.0))

    part = _edge_kernel(hsd.reshape(2 * N, D), idx, zrows, tinit)

    return _tc_post(
        h, part, w2i, b2i.reshape(1, D), wn1[:D], wn1[D:], bn1.reshape(1, D),
        wn2, bn2.reshape(1, D), wo1, bo1.reshape(1, D), wo2, bo2.reshape(1, NC))
